# baseline (speedup 1.0000x reference)
import jax
import jax.numpy as jnp
from jax import lax
from jax.experimental import pallas as pl
from jax.experimental.pallas import tpu as pltpu
from jax.experimental.pallas import tpu_sc as plsc

_B = 16384
_D = 128
_NC = 2
_NS = 16
_NW = _NC * _NS
_LANES = 16
_CHUNK = 128
_PER_W = _B // _NW
_NCHUNK = _PER_W // _CHUNK


def _sc_body(i_hbm, j_hbm, k_hbm, u_hbm, v_hbm, out_hbm,
             ib, jb, kb, u_buf, vj_buf, vk_buf, out_buf,
             sem_u, sem_j, sem_k):
    wid = lax.axis_index("s") * _NC + lax.axis_index("c")
    pltpu.sync_copy(i_hbm.at[wid], ib)
    pltpu.sync_copy(j_hbm.at[wid], jb)
    pltpu.sync_copy(k_hbm.at[wid], kb)
    for c in range(_NCHUNK):
        cu = pltpu.async_copy(u_hbm.at[ib.at[c]], u_buf, sem_u)
        cj = pltpu.async_copy(v_hbm.at[jb.at[c]], vj_buf, sem_j)
        ck = pltpu.async_copy(v_hbm.at[kb.at[c]], vk_buf, sem_k)
        cu.wait()
        cj.wait()
        ck.wait()
        for g in range(_CHUNK // _LANES):
            rows = lax.iota(jnp.int32, _LANES) + (g * _LANES)

            def dbody(d8, acc, rows=rows):
                for dd in range(8):
                    dcol = jnp.zeros((_LANES,), jnp.int32) + (d8 * 8 + dd)
                    uc = plsc.load_gather(u_buf, [rows, dcol])
                    vjc = plsc.load_gather(vj_buf, [rows, dcol])
                    vkc = plsc.load_gather(vk_buf, [rows, dcol])
                    acc = acc + uc * (vjc - vkc)
                return acc

            acc = lax.fori_loop(0, _D // 8, dbody,
                                jnp.zeros((_LANES,), jnp.float32))
            sig = 1.0 / (1.0 + jnp.exp(-acc))
            out_buf[pl.ds(g * _LANES, _LANES)] = sig
        pltpu.sync_copy(out_buf, out_hbm.at[wid, c])


@jax.jit
def kernel(i, j, k, u_weight, v_weight):
    i3 = i.reshape(_NW, _NCHUNK, _CHUNK)
    j3 = j.reshape(_NW, _NCHUNK, _CHUNK)
    k3 = k.reshape(_NW, _NCHUNK, _CHUNK)
    run = pl.kernel(
        _sc_body,
        out_type=jax.ShapeDtypeStruct((_NW, _NCHUNK, _CHUNK), jnp.float32),
        mesh=plsc.VectorSubcoreMesh(core_axis_name="c", subcore_axis_name="s"),
        scratch_types=[
            pltpu.VMEM((_NCHUNK, _CHUNK), jnp.int32),
            pltpu.VMEM((_NCHUNK, _CHUNK), jnp.int32),
            pltpu.VMEM((_NCHUNK, _CHUNK), jnp.int32),
            pltpu.VMEM((_CHUNK, _D), jnp.float32),
            pltpu.VMEM((_CHUNK, _D), jnp.float32),
            pltpu.VMEM((_CHUNK, _D), jnp.float32),
            pltpu.VMEM((_CHUNK,), jnp.float32),
            pltpu.SemaphoreType.DMA,
            pltpu.SemaphoreType.DMA,
            pltpu.SemaphoreType.DMA,
        ],
        compiler_params=pltpu.CompilerParams(needs_layout_passes=False),
    )
    out = run(i3, j3, k3, u_weight, v_weight)
    return out.reshape(_B)


# speedup vs baseline: 3.1068x; 3.1068x over previous
import jax
import jax.numpy as jnp
from jax import lax
from jax.experimental import pallas as pl
from jax.experimental.pallas import tpu as pltpu
from jax.experimental.pallas import tpu_sc as plsc

_B = 16384
_D = 128
_NC = 2
_NS = 16
_NW = _NC * _NS
_LANES = 16
_CHUNK = 128
_PER_W = _B // _NW
_NCHUNK = _PER_W // _CHUNK


def _sc_body(i_hbm, j_hbm, k_hbm, u_hbm, v_hbm, out_hbm,
             ib, jb, kb, u_buf0, u_buf1, vj_buf0, vj_buf1,
             vk_buf0, vk_buf1, out_buf, sem0, sem1):
    wid = lax.axis_index("s") * _NC + lax.axis_index("c")
    pltpu.sync_copy(i_hbm.at[wid], ib)
    pltpu.sync_copy(j_hbm.at[wid], jb)
    pltpu.sync_copy(k_hbm.at[wid], kb)
    ubufs = (u_buf0, u_buf1)
    jbufs = (vj_buf0, vj_buf1)
    kbufs = (vk_buf0, vk_buf1)
    sems = (sem0, sem1)

    def start(c):
        p = c % 2
        return (pltpu.async_copy(u_hbm.at[ib.at[c]], ubufs[p], sems[p]),
                pltpu.async_copy(v_hbm.at[jb.at[c]], jbufs[p], sems[p]),
                pltpu.async_copy(v_hbm.at[kb.at[c]], kbufs[p], sems[p]))

    lane = lax.iota(jnp.int32, _LANES)
    cur = start(0)
    for c in range(_NCHUNK):
        p = c % 2
        nxt = start(c + 1) if c + 1 < _NCHUNK else None
        for cp in cur:
            cp.wait()
        ub, jbf, kbf = ubufs[p], jbufs[p], kbufs[p]
        for g in range(_CHUNK // _LANES):
            rows = lane + (g * _LANES)

            def dbody(d8, acc, rows=rows):
                for dd in range(8):
                    dcol = (lane + (d8 * 8 + dd)) & (_D - 1)
                    uc = plsc.load_gather(ub, [rows, dcol])
                    vjc = plsc.load_gather(jbf, [rows, dcol])
                    vkc = plsc.load_gather(kbf, [rows, dcol])
                    acc = acc + uc * (vjc - vkc)
                return acc

            acc = lax.fori_loop(0, _D // 8, dbody,
                                jnp.zeros((_LANES,), jnp.float32))
            sig = 1.0 / (1.0 + jnp.exp(-acc))
            out_buf[pl.ds(g * _LANES, _LANES)] = sig
        pltpu.sync_copy(out_buf, out_hbm.at[wid, c])
        cur = nxt


@jax.jit
def kernel(i, j, k, u_weight, v_weight):
    i3 = i.reshape(_NW, _NCHUNK, _CHUNK)
    j3 = j.reshape(_NW, _NCHUNK, _CHUNK)
    k3 = k.reshape(_NW, _NCHUNK, _CHUNK)
    run = pl.kernel(
        _sc_body,
        out_type=jax.ShapeDtypeStruct((_NW, _NCHUNK, _CHUNK), jnp.float32),
        mesh=plsc.VectorSubcoreMesh(core_axis_name="c", subcore_axis_name="s"),
        scratch_types=[
            pltpu.VMEM((_NCHUNK, _CHUNK), jnp.int32),
            pltpu.VMEM((_NCHUNK, _CHUNK), jnp.int32),
            pltpu.VMEM((_NCHUNK, _CHUNK), jnp.int32),
            pltpu.VMEM((_CHUNK, _D), jnp.float32),
            pltpu.VMEM((_CHUNK, _D), jnp.float32),
            pltpu.VMEM((_CHUNK, _D), jnp.float32),
            pltpu.VMEM((_CHUNK, _D), jnp.float32),
            pltpu.VMEM((_CHUNK, _D), jnp.float32),
            pltpu.VMEM((_CHUNK, _D), jnp.float32),
            pltpu.VMEM((_CHUNK,), jnp.float32),
            pltpu.SemaphoreType.DMA,
            pltpu.SemaphoreType.DMA,
        ],
        compiler_params=pltpu.CompilerParams(needs_layout_passes=False),
    )
    out = run(i3, j3, k3, u_weight, v_weight)
    return out.reshape(_B)
